# baseline (speedup 1.0000x reference)
import functools

import jax
import jax.numpy as jnp
from jax import lax
from jax.experimental import pallas as pl
from jax.experimental.pallas import tpu as pltpu
from jax.experimental.pallas import tpu_sc as plsc

_NC, _NS = 2, 16
_SCCH = 80
_DEPTH = 3


def _sc_mesh():
    return plsc.VectorSubcoreMesh(core_axis_name="c", subcore_axis_name="s",
                                  num_cores=_NC, num_subcores=_NS)


@functools.lru_cache(maxsize=None)
def _make_scatter_add(n_edges: int, n_nodes: int, h: int,
                      dst_off: int = 0, init_acc: bool = False):
    hh = h // 2
    ept = n_edges // _NS
    scch = _SCCH if ept % _SCCH == 0 else _SCCH // 2
    chunks = ept // scch
    assert ept % scch == 0 and chunks % 2 == 1 and chunks >= 3
    stripe = 1000
    n_stripes = n_nodes // stripe

    @functools.partial(
        pl.kernel,
        out_type=jax.ShapeDtypeStruct((n_nodes, h), jnp.float32),
        mesh=_sc_mesh(),
        scratch_types=[
            pltpu.VMEM((ept,), jnp.int32),
            pltpu.VMEM((scch, hh), jnp.float32),
            pltpu.VMEM((scch, hh), jnp.float32),
            pltpu.VMEM_SHARED((n_nodes, hh), jnp.float32),
            pltpu.SemaphoreType.DMA,
            pltpu.SemaphoreType.DMA,
            pltpu.SemaphoreType.DMA,
            pltpu.SemaphoreType.DMA,
        ],
    )
    def scatter_kernel(msg_h, dst_h, init_h, out_h, idx_v, buf_v, buf_w,
                       acc_sh, l0, l1, s0, s1):
        c = lax.axis_index("c")
        t = lax.axis_index("s")

        @pl.when(t < n_stripes)
        def _():
            if init_acc:
                pltpu.sync_copy(
                    init_h.at[pl.ds(t * stripe, stripe), pl.ds(c * hh, hh)],
                    acc_sh.at[pl.ds(t * stripe, stripe)])
            else:
                pltpu.sync_copy(init_h, acc_sh.at[pl.ds(t * stripe, stripe)])

        pltpu.sync_copy(dst_h.at[pl.ds(dst_off + t * ept, ept)], idx_v)
        plsc.subcore_barrier()

        bufs = ((buf_v, l0, s0), (buf_w, l1, s1))

        def eslice(j):
            return (pl.ds(t * ept + j * scch, scch), pl.ds(c * hh, hh))

        def a_load(j, buf, sem):
            pltpu.async_copy(msg_h.at[eslice(j)], buf, sem)

        def a_load_wait(j, buf, sem):
            pltpu.make_async_copy(msg_h.at[eslice(j)], buf, sem).wait()

        def a_scat(j, buf, sem):
            pltpu.async_copy(buf, acc_sh.at[idx_v.at[pl.ds(j * scch, scch)]],
                             sem, add=True)

        def a_scat_wait(j, buf, sem):
            pltpu.make_async_copy(
                buf, acc_sh.at[idx_v.at[pl.ds(j * scch, scch)]], sem).wait()

        def a_step(j, b):
            buf, lsem, ssem = bufs[b]
            nbuf, nlsem, nssem = bufs[1 - b]

            @pl.when(j >= 1)
            def _():
                a_scat_wait(j - 1, nbuf, nssem)

            @pl.when(j + 1 < chunks)
            def _():
                a_load(j + 1, nbuf, nlsem)

            a_load_wait(j, buf, lsem)
            a_scat(j, buf, ssem)

        a_load(0, buf_v, l0)

        def a_outer(i, carry):
            a_step(i * 2, 0)
            a_step(i * 2 + 1, 1)
            return carry

        lax.fori_loop(0, chunks // 2, a_outer, 0)
        a_scat_wait(chunks - 2, buf_w, s1)
        a_load_wait(chunks - 1, buf_v, l0)
        a_scat(chunks - 1, buf_v, s0)
        a_scat_wait(chunks - 1, buf_v, s0)
        plsc.subcore_barrier()

        @pl.when(t < n_stripes)
        def _():
            pltpu.sync_copy(
                acc_sh.at[pl.ds(t * stripe, stripe)],
                out_h.at[pl.ds(t * stripe, stripe), pl.ds(c * hh, hh)])

    return scatter_kernel


@functools.lru_cache(maxsize=None)
def _make_mp_step(scat_edges: int, gath_edges: int, n_nodes: int, h: int,
                  dst_off: int = 0, init_acc: bool = False):
    hh = h // 2
    sept = scat_edges // _NS
    scch = _SCCH if sept % _SCCH == 0 else _SCCH // 2
    schunks = sept // scch
    assert sept % scch == 0 and schunks % 2 == 1 and schunks >= 3
    gept = gath_edges // _NS
    gchunks = gept // _SCCH
    assert gept % _SCCH == 0 and gchunks % 2 == 1 and gchunks >= 3
    stripe = 1000
    n_stripes = n_nodes // stripe

    shared_bufs = scch == _SCCH
    scratch = [
        pltpu.VMEM((sept,), jnp.int32),
        pltpu.VMEM((gept,), jnp.int32),
        pltpu.VMEM((scch, hh), jnp.float32),
        pltpu.VMEM((scch, hh), jnp.float32),
    ]
    if not shared_bufs:
        scratch += [pltpu.VMEM((_SCCH, hh), jnp.float32),
                    pltpu.VMEM((_SCCH, hh), jnp.float32)]
    scratch += [
        pltpu.VMEM_SHARED((n_nodes, hh), jnp.float32),
        pltpu.SemaphoreType.DMA,
        pltpu.SemaphoreType.DMA,
        pltpu.SemaphoreType.DMA,
        pltpu.SemaphoreType.DMA,
    ]

    @functools.partial(
        pl.kernel,
        out_type=jax.ShapeDtypeStruct((gath_edges, h), jnp.float32),
        mesh=_sc_mesh(),
        scratch_types=scratch,
    )
    def mp_kernel(msg_h, dst_h, src_h, init_h, g_h,
                  dst_v, src_v, sb0, sb1, *rest):
        if shared_bufs:
            gb0, gb1 = sb0, sb1
            acc_sh, l0, l1, s0, s1 = rest
        else:
            gb0, gb1, acc_sh, l0, l1, s0, s1 = rest
        c = lax.axis_index("c")
        t = lax.axis_index("s")

        @pl.when(t < n_stripes)
        def _():
            if init_acc:
                pltpu.sync_copy(
                    init_h.at[pl.ds(t * stripe, stripe), pl.ds(c * hh, hh)],
                    acc_sh.at[pl.ds(t * stripe, stripe)])
            else:
                pltpu.sync_copy(init_h, acc_sh.at[pl.ds(t * stripe, stripe)])

        pltpu.sync_copy(dst_h.at[pl.ds(dst_off + t * sept, sept)], dst_v)
        pltpu.sync_copy(src_h.at[pl.ds(t * gept, gept)], src_v)
        plsc.subcore_barrier()

        sbufs = ((sb0, l0, s0), (sb1, l1, s1))

        def sslice(j):
            return (pl.ds(t * sept + j * scch, scch), pl.ds(c * hh, hh))

        def a_load(j, buf, sem):
            pltpu.async_copy(msg_h.at[sslice(j)], buf, sem)

        def a_load_wait(j, buf, sem):
            pltpu.make_async_copy(msg_h.at[sslice(j)], buf, sem).wait()

        def a_scat(j, buf, sem):
            pltpu.async_copy(buf, acc_sh.at[dst_v.at[pl.ds(j * scch, scch)]],
                             sem, add=True)

        def a_scat_wait(j, buf, sem):
            pltpu.make_async_copy(
                buf, acc_sh.at[dst_v.at[pl.ds(j * scch, scch)]], sem).wait()

        def a_step(j, b):
            buf, lsem, ssem = sbufs[b]
            nbuf, nlsem, nssem = sbufs[1 - b]

            @pl.when(j >= 1)
            def _():
                a_scat_wait(j - 1, nbuf, nssem)

            @pl.when(j + 1 < schunks)
            def _():
                a_load(j + 1, nbuf, nlsem)

            a_load_wait(j, buf, lsem)
            a_scat(j, buf, ssem)

        a_load(0, sb0, l0)

        def a_outer(i, carry):
            a_step(i * 2, 0)
            a_step(i * 2 + 1, 1)
            return carry

        lax.fori_loop(0, schunks // 2, a_outer, 0)
        a_scat_wait(schunks - 2, sb1, s1)
        a_load_wait(schunks - 1, sb0, l0)
        a_scat(schunks - 1, sb0, s0)
        a_scat_wait(schunks - 1, sb0, s0)
        plsc.subcore_barrier()

        gbufs = ((gb0, l0, s0), (gb1, l1, s1))

        def gslice(j):
            return (pl.ds(t * gept + j * _SCCH, _SCCH), pl.ds(c * hh, hh))

        def b_gat(j, buf, sem):
            pltpu.async_copy(acc_sh.at[src_v.at[pl.ds(j * _SCCH, _SCCH)]],
                             buf, sem)

        def b_gat_wait(j, buf, sem):
            pltpu.make_async_copy(
                acc_sh.at[src_v.at[pl.ds(j * _SCCH, _SCCH)]], buf, sem).wait()

        def b_out(j, buf, sem):
            pltpu.async_copy(buf, g_h.at[gslice(j)], sem)

        def b_out_wait(j, buf, sem):
            pltpu.make_async_copy(buf, g_h.at[gslice(j)], sem).wait()

        def b_step(j, b):
            buf, gsem, osem = gbufs[b]
            nbuf, ngsem, nosem = gbufs[1 - b]

            @pl.when(j >= 1)
            def _():
                b_out_wait(j - 1, nbuf, nosem)

            @pl.when(j + 1 < gchunks)
            def _():
                b_gat(j + 1, nbuf, ngsem)

            b_gat_wait(j, buf, gsem)
            b_out(j, buf, osem)

        b_gat(0, gb0, l0)

        def b_outer(i, carry):
            b_step(i * 2, 0)
            b_step(i * 2 + 1, 1)
            return carry

        lax.fori_loop(0, gchunks // 2, b_outer, 0)
        b_out_wait(gchunks - 2, gb1, s1)
        b_gat_wait(gchunks - 1, gb0, l0)
        b_out(gchunks - 1, gb0, s0)
        b_out_wait(gchunks - 1, gb0, s0)

    return mp_kernel


def _dotT(xt, w):
    return lax.dot_general(xt, w, dimension_numbers=(((0,), (0,)), ((), ())),
                           preferred_element_type=jnp.float32)


def _mm_body(xt_ref, w_ref, o_ref):
    o_ref[...] = _dotT(xt_ref[...], w_ref[...]).astype(jnp.bfloat16)


def _mm_relu_body(xt_ref, w_ref, o_ref):
    o_ref[...] = jnp.maximum(_dotT(xt_ref[...], w_ref[...]), 0.0)


def _matmul(fbT, w_i, bm, relu):
    k, e = fbT.shape
    _, h = w_i.shape
    return pl.pallas_call(
        _mm_relu_body if relu else _mm_body,
        grid=(e // bm,),
        in_specs=[pl.BlockSpec((k, bm), lambda i: (0, i)),
                  pl.BlockSpec((k, h), lambda i: (0, 0))],
        out_specs=pl.BlockSpec((bm, h), lambda i: (i, 0)),
        out_shape=jax.ShapeDtypeStruct(
            (e, h), jnp.float32 if relu else jnp.bfloat16),
    )(fbT, w_i)


def _fused_iter_body(msg_ref, inp_ref, g_ref, wh_ref, o_ref):
    m = msg_ref[...]
    up = jnp.roll(m, -1, axis=0)
    dn = jnp.roll(m, 1, axis=0)
    parity = lax.broadcasted_iota(jnp.int32, m.shape, 0) % 2
    m_swapped = jnp.where(parity == 0, up, dn)
    p = jnp.dot(g_ref[...] - m_swapped, wh_ref[...],
                preferred_element_type=jnp.float32)
    o_ref[...] = jnp.maximum(inp_ref[...].astype(jnp.float32) + p, 0.0)


def _fused_iter(msg, inp, g, w_h, bm, half, e_out):
    e_msg, h = msg.shape
    off = half * (e_out // bm) if e_msg != e_out else 0
    goff = half * (e_out // bm)
    return pl.pallas_call(
        _fused_iter_body,
        grid=(e_out // bm,),
        in_specs=[pl.BlockSpec((bm, h), lambda i: (i + off, 0)),
                  pl.BlockSpec((bm, h), lambda i: (i + goff, 0)),
                  pl.BlockSpec((bm, h), lambda i: (i + goff, 0)),
                  pl.BlockSpec((h, h), lambda i: (0, 0))],
        out_specs=pl.BlockSpec((bm, h), lambda i: (i, 0)),
        out_shape=jax.ShapeDtypeStruct((e_out, h), jnp.float32),
    )(msg, inp, g, w_h)


def _final_body(fa_ref, am_ref, w1_ref, w2_ref, b_ref, o_ref):
    acc = jnp.dot(fa_ref[...], w1_ref[...], preferred_element_type=jnp.float32)
    acc += jnp.dot(am_ref[...], w2_ref[...], preferred_element_type=jnp.float32)
    o_ref[...] = jnp.maximum(acc + b_ref[...], 0.0)


def _final_atoms(f_atoms, a_msg, w_o1, w_o2, b_o, bm):
    n, ka = f_atoms.shape
    _, h = w_o1.shape
    return pl.pallas_call(
        _final_body,
        grid=(n // bm,),
        in_specs=[pl.BlockSpec((bm, ka), lambda i: (i, 0)),
                  pl.BlockSpec((bm, h), lambda i: (i, 0)),
                  pl.BlockSpec((ka, h), lambda i: (0, 0)),
                  pl.BlockSpec((h, h), lambda i: (0, 0)),
                  pl.BlockSpec((1, h), lambda i: (0, 0))],
        out_specs=pl.BlockSpec((bm, h), lambda i: (i, 0)),
        out_shape=jax.ShapeDtypeStruct((n, h), jnp.float32),
    )(f_atoms, a_msg, w_o1, w_o2, b_o)


def _sys_body(s_ref, w_ref, b_ref, o_ref):
    o_ref[...] = jnp.dot(s_ref[...], w_ref[...],
                         preferred_element_type=jnp.float32) + b_ref[...]


def _sys_emb(sysf, w_s, b_s):
    b, k = sysf.shape
    _, h = w_s.shape
    return pl.pallas_call(
        _sys_body,
        in_specs=[pl.BlockSpec((b, k), lambda: (0, 0)),
                  pl.BlockSpec((k, h), lambda: (0, 0)),
                  pl.BlockSpec((1, h), lambda: (0, 0))],
        out_specs=pl.BlockSpec((b, h), lambda: (0, 0)),
        out_shape=jax.ShapeDtypeStruct((b, h), jnp.float32),
    )(sysf, w_s, b_s)


def kernel(f_atoms, f_bonds, edge_index, sysf, W_i, W_h, W_o, b_o, W_s, b_s, pad_token):
    n, atom_f = f_atoms.shape
    e = f_bonds.shape[0]
    h = W_i.shape[1]
    b = sysf.shape[0]
    s = n // b

    src = edge_index[0]
    dst = edge_index[1]
    zeros = jnp.zeros((1000, h // 2), jnp.float32)
    e2 = e // 2

    mp_full = _make_mp_step(e, e, n, h)
    scat_h1 = _make_scatter_add(e2, n, h, dst_off=0, init_acc=False)
    mp_h2 = _make_mp_step(e2, e, n, h, dst_off=e2, init_acc=True)
    scat_last = _make_scatter_add(e2, n, h, dst_off=e2, init_acc=True)

    fbT = f_bonds.T
    msg = _matmul(fbT, W_i, bm=3200, relu=True)
    inp = _matmul(fbT, W_i, bm=3200, relu=False)

    g = mp_full(msg, dst, src, zeros)
    m1 = _fused_iter(msg, inp, g, W_h, 1600, 0, e2)
    m2 = _fused_iter(msg, inp, g, W_h, 1600, 1, e2)
    accp = scat_h1(m1, dst, zeros)
    g = mp_h2(m2, dst, src, accp)
    m1 = _fused_iter(m1, inp, g, W_h, 1600, 0, e2)
    m2 = _fused_iter(m2, inp, g, W_h, 1600, 1, e2)
    accp = scat_h1(m1, dst, zeros)
    a_msg = scat_last(m2, dst, accp)

    atoms = _final_atoms(f_atoms, a_msg, W_o[:atom_f], W_o[atom_f:],
                         b_o[None, :], bm=1000)
    sys_out = _sys_emb(sysf, W_s, b_s[None, :])
    return (sys_out[:, None, :], atoms.reshape(b, s, h))


# speedup vs baseline: 1.0667x; 1.0667x over previous
import functools

import jax
import jax.numpy as jnp
from jax import lax
from jax.experimental import pallas as pl
from jax.experimental.pallas import tpu as pltpu
from jax.experimental.pallas import tpu_sc as plsc

_NC, _NS = 2, 16
_SCCH = 80
_DEPTH = 3


def _sc_mesh():
    return plsc.VectorSubcoreMesh(core_axis_name="c", subcore_axis_name="s",
                                  num_cores=_NC, num_subcores=_NS)


@functools.lru_cache(maxsize=None)
def _make_scatter_add(n_edges: int, n_nodes: int, h: int,
                      dst_off: int = 0, init_acc: bool = False):
    hh = h // 2
    ept = n_edges // _NS
    scch = _SCCH if ept % _SCCH == 0 else _SCCH // 2
    chunks = ept // scch
    assert ept % scch == 0 and chunks >= 3
    stripe = 1000
    n_stripes = n_nodes // stripe

    @functools.partial(
        pl.kernel,
        out_type=jax.ShapeDtypeStruct((n_nodes, h), jnp.float32),
        mesh=_sc_mesh(),
        scratch_types=[
            pltpu.VMEM((ept,), jnp.int32),
            pltpu.VMEM((scch, hh), jnp.float32),
            pltpu.VMEM((scch, hh), jnp.float32),
            pltpu.VMEM_SHARED((n_nodes, hh), jnp.float32),
            pltpu.SemaphoreType.DMA,
            pltpu.SemaphoreType.DMA,
            pltpu.SemaphoreType.DMA,
            pltpu.SemaphoreType.DMA,
        ],
    )
    def scatter_kernel(msg_h, dst_h, init_h, out_h, idx_v, buf_v, buf_w,
                       acc_sh, l0, l1, s0, s1):
        c = lax.axis_index("c")
        t = lax.axis_index("s")

        @pl.when(t < n_stripes)
        def _():
            if init_acc:
                pltpu.sync_copy(
                    init_h.at[pl.ds(t * stripe, stripe), pl.ds(c * hh, hh)],
                    acc_sh.at[pl.ds(t * stripe, stripe)])
            else:
                pltpu.sync_copy(init_h, acc_sh.at[pl.ds(t * stripe, stripe)])

        pltpu.sync_copy(dst_h.at[pl.ds(dst_off + t * ept, ept)], idx_v)
        plsc.subcore_barrier()

        bufs = ((buf_v, l0, s0), (buf_w, l1, s1))

        def eslice(j):
            return (pl.ds(t * ept + j * scch, scch), pl.ds(c * hh, hh))

        def a_load(j, buf, sem):
            pltpu.async_copy(msg_h.at[eslice(j)], buf, sem)

        def a_load_wait(j, buf, sem):
            pltpu.make_async_copy(msg_h.at[eslice(j)], buf, sem).wait()

        def a_scat(j, buf, sem):
            pltpu.async_copy(buf, acc_sh.at[idx_v.at[pl.ds(j * scch, scch)]],
                             sem, add=True)

        def a_scat_wait(j, buf, sem):
            pltpu.make_async_copy(
                buf, acc_sh.at[idx_v.at[pl.ds(j * scch, scch)]], sem).wait()

        def a_step(j, b):
            buf, lsem, ssem = bufs[b]
            nbuf, nlsem, nssem = bufs[1 - b]

            @pl.when(j >= 1)
            def _():
                a_scat_wait(j - 1, nbuf, nssem)

            @pl.when(j + 1 < chunks)
            def _():
                a_load(j + 1, nbuf, nlsem)

            a_load_wait(j, buf, lsem)
            a_scat(j, buf, ssem)

        a_load(0, buf_v, l0)

        def a_outer(i, carry):
            a_step(i * 2, 0)
            a_step(i * 2 + 1, 1)
            return carry

        lax.fori_loop(0, chunks // 2, a_outer, 0)
        if chunks % 2:
            a_scat_wait(chunks - 2, buf_w, s1)
            a_load_wait(chunks - 1, buf_v, l0)
            a_scat(chunks - 1, buf_v, s0)
            a_scat_wait(chunks - 1, buf_v, s0)
        else:
            a_scat_wait(chunks - 1, buf_w, s1)
        plsc.subcore_barrier()

        @pl.when(t < n_stripes)
        def _():
            pltpu.sync_copy(
                acc_sh.at[pl.ds(t * stripe, stripe)],
                out_h.at[pl.ds(t * stripe, stripe), pl.ds(c * hh, hh)])

    return scatter_kernel


@functools.lru_cache(maxsize=None)
def _make_mp_step(scat_edges: int, gath_edges: int, n_nodes: int, h: int,
                  dst_off: int = 0, init_acc: bool = False):
    hh = h // 2
    sept = scat_edges // _NS
    scch = _SCCH if sept % _SCCH == 0 else _SCCH // 2
    schunks = sept // scch
    assert sept % scch == 0 and schunks >= 3
    gept = gath_edges // _NS
    gchunks = gept // _SCCH
    assert gept % _SCCH == 0 and gchunks >= 3
    stripe = 1000
    n_stripes = n_nodes // stripe

    shared_bufs = scch == _SCCH
    scratch = [
        pltpu.VMEM((sept,), jnp.int32),
        pltpu.VMEM((gept,), jnp.int32),
        pltpu.VMEM((scch, hh), jnp.float32),
        pltpu.VMEM((scch, hh), jnp.float32),
    ]
    if not shared_bufs:
        scratch += [pltpu.VMEM((_SCCH, hh), jnp.float32),
                    pltpu.VMEM((_SCCH, hh), jnp.float32)]
    scratch += [
        pltpu.VMEM_SHARED((n_nodes, hh), jnp.float32),
        pltpu.SemaphoreType.DMA,
        pltpu.SemaphoreType.DMA,
        pltpu.SemaphoreType.DMA,
        pltpu.SemaphoreType.DMA,
    ]

    @functools.partial(
        pl.kernel,
        out_type=jax.ShapeDtypeStruct((gath_edges, h), jnp.float32),
        mesh=_sc_mesh(),
        scratch_types=scratch,
    )
    def mp_kernel(msg_h, dst_h, src_h, init_h, g_h,
                  dst_v, src_v, sb0, sb1, *rest):
        if shared_bufs:
            gb0, gb1 = sb0, sb1
            acc_sh, l0, l1, s0, s1 = rest
        else:
            gb0, gb1, acc_sh, l0, l1, s0, s1 = rest
        c = lax.axis_index("c")
        t = lax.axis_index("s")

        @pl.when(t < n_stripes)
        def _():
            if init_acc:
                pltpu.sync_copy(
                    init_h.at[pl.ds(t * stripe, stripe), pl.ds(c * hh, hh)],
                    acc_sh.at[pl.ds(t * stripe, stripe)])
            else:
                pltpu.sync_copy(init_h, acc_sh.at[pl.ds(t * stripe, stripe)])

        pltpu.sync_copy(dst_h.at[pl.ds(dst_off + t * sept, sept)], dst_v)
        pltpu.sync_copy(src_h.at[pl.ds(t * gept, gept)], src_v)
        plsc.subcore_barrier()

        sbufs = ((sb0, l0, s0), (sb1, l1, s1))

        def sslice(j):
            return (pl.ds(t * sept + j * scch, scch), pl.ds(c * hh, hh))

        def a_load(j, buf, sem):
            pltpu.async_copy(msg_h.at[sslice(j)], buf, sem)

        def a_load_wait(j, buf, sem):
            pltpu.make_async_copy(msg_h.at[sslice(j)], buf, sem).wait()

        def a_scat(j, buf, sem):
            pltpu.async_copy(buf, acc_sh.at[dst_v.at[pl.ds(j * scch, scch)]],
                             sem, add=True)

        def a_scat_wait(j, buf, sem):
            pltpu.make_async_copy(
                buf, acc_sh.at[dst_v.at[pl.ds(j * scch, scch)]], sem).wait()

        def a_step(j, b):
            buf, lsem, ssem = sbufs[b]
            nbuf, nlsem, nssem = sbufs[1 - b]

            @pl.when(j >= 1)
            def _():
                a_scat_wait(j - 1, nbuf, nssem)

            @pl.when(j + 1 < schunks)
            def _():
                a_load(j + 1, nbuf, nlsem)

            a_load_wait(j, buf, lsem)
            a_scat(j, buf, ssem)

        a_load(0, sb0, l0)

        def a_outer(i, carry):
            a_step(i * 2, 0)
            a_step(i * 2 + 1, 1)
            return carry

        lax.fori_loop(0, schunks // 2, a_outer, 0)
        if schunks % 2:
            a_scat_wait(schunks - 2, sb1, s1)
            a_load_wait(schunks - 1, sb0, l0)
            a_scat(schunks - 1, sb0, s0)
            a_scat_wait(schunks - 1, sb0, s0)
        else:
            a_scat_wait(schunks - 1, sb1, s1)
        plsc.subcore_barrier()

        gbufs = ((gb0, l0, s0), (gb1, l1, s1))

        def gslice(j):
            return (pl.ds(t * gept + j * _SCCH, _SCCH), pl.ds(c * hh, hh))

        def b_gat(j, buf, sem):
            pltpu.async_copy(acc_sh.at[src_v.at[pl.ds(j * _SCCH, _SCCH)]],
                             buf, sem)

        def b_gat_wait(j, buf, sem):
            pltpu.make_async_copy(
                acc_sh.at[src_v.at[pl.ds(j * _SCCH, _SCCH)]], buf, sem).wait()

        def b_out(j, buf, sem):
            pltpu.async_copy(buf, g_h.at[gslice(j)], sem)

        def b_out_wait(j, buf, sem):
            pltpu.make_async_copy(buf, g_h.at[gslice(j)], sem).wait()

        def b_step(j, b):
            buf, gsem, osem = gbufs[b]
            nbuf, ngsem, nosem = gbufs[1 - b]

            @pl.when(j >= 1)
            def _():
                b_out_wait(j - 1, nbuf, nosem)

            @pl.when(j + 1 < gchunks)
            def _():
                b_gat(j + 1, nbuf, ngsem)

            b_gat_wait(j, buf, gsem)
            b_out(j, buf, osem)

        b_gat(0, gb0, l0)

        def b_outer(i, carry):
            b_step(i * 2, 0)
            b_step(i * 2 + 1, 1)
            return carry

        lax.fori_loop(0, gchunks // 2, b_outer, 0)
        if gchunks % 2:
            b_out_wait(gchunks - 2, gb1, s1)
            b_gat_wait(gchunks - 1, gb0, l0)
            b_out(gchunks - 1, gb0, s0)
            b_out_wait(gchunks - 1, gb0, s0)
        else:
            b_out_wait(gchunks - 1, gb1, s1)

    return mp_kernel


def _dotT(xt, w):
    return lax.dot_general(xt, w, dimension_numbers=(((0,), (0,)), ((), ())),
                           preferred_element_type=jnp.float32)


def _mm_body(xt_ref, w_ref, o_ref):
    o_ref[...] = _dotT(xt_ref[...], w_ref[...]).astype(jnp.bfloat16)


def _mm_relu_body(xt_ref, w_ref, o_ref):
    o_ref[...] = jnp.maximum(_dotT(xt_ref[...], w_ref[...]), 0.0)


def _matmul(fbT, w_i, bm, relu):
    k, e = fbT.shape
    _, h = w_i.shape
    return pl.pallas_call(
        _mm_relu_body if relu else _mm_body,
        grid=(e // bm,),
        in_specs=[pl.BlockSpec((k, bm), lambda i: (0, i)),
                  pl.BlockSpec((k, h), lambda i: (0, 0))],
        out_specs=pl.BlockSpec((bm, h), lambda i: (i, 0)),
        out_shape=jax.ShapeDtypeStruct(
            (e, h), jnp.float32 if relu else jnp.bfloat16),
    )(fbT, w_i)


def _fused_iter_body(msg_ref, inp_ref, g_ref, wh_ref, o_ref):
    m = msg_ref[...]
    up = jnp.roll(m, -1, axis=0)
    dn = jnp.roll(m, 1, axis=0)
    parity = lax.broadcasted_iota(jnp.int32, m.shape, 0) % 2
    m_swapped = jnp.where(parity == 0, up, dn)
    p = jnp.dot(g_ref[...] - m_swapped, wh_ref[...],
                preferred_element_type=jnp.float32)
    o_ref[...] = jnp.maximum(inp_ref[...].astype(jnp.float32) + p, 0.0)


def _fused_iter(msg, inp, g, w_h, bm, part, e_a, e_b=None):
    e_out = e_a if part == 0 else e_b
    e_msg, h = msg.shape
    goff = part * (e_a // bm)
    off = goff if e_msg != e_out else 0
    return pl.pallas_call(
        _fused_iter_body,
        grid=(e_out // bm,),
        in_specs=[pl.BlockSpec((bm, h), lambda i: (i + off, 0)),
                  pl.BlockSpec((bm, h), lambda i: (i + goff, 0)),
                  pl.BlockSpec((bm, h), lambda i: (i + goff, 0)),
                  pl.BlockSpec((h, h), lambda i: (0, 0))],
        out_specs=pl.BlockSpec((bm, h), lambda i: (i, 0)),
        out_shape=jax.ShapeDtypeStruct((e_out, h), jnp.float32),
    )(msg, inp, g, w_h)


def _final_body(fa_ref, am_ref, w1_ref, w2_ref, b_ref, o_ref):
    acc = jnp.dot(fa_ref[...], w1_ref[...], preferred_element_type=jnp.float32)
    acc += jnp.dot(am_ref[...], w2_ref[...], preferred_element_type=jnp.float32)
    o_ref[...] = jnp.maximum(acc + b_ref[...], 0.0)


def _final_atoms(f_atoms, a_msg, w_o1, w_o2, b_o, bm):
    n, ka = f_atoms.shape
    _, h = w_o1.shape
    return pl.pallas_call(
        _final_body,
        grid=(n // bm,),
        in_specs=[pl.BlockSpec((bm, ka), lambda i: (i, 0)),
                  pl.BlockSpec((bm, h), lambda i: (i, 0)),
                  pl.BlockSpec((ka, h), lambda i: (0, 0)),
                  pl.BlockSpec((h, h), lambda i: (0, 0)),
                  pl.BlockSpec((1, h), lambda i: (0, 0))],
        out_specs=pl.BlockSpec((bm, h), lambda i: (i, 0)),
        out_shape=jax.ShapeDtypeStruct((n, h), jnp.float32),
    )(f_atoms, a_msg, w_o1, w_o2, b_o)


def _sys_body(s_ref, w_ref, b_ref, o_ref):
    o_ref[...] = jnp.dot(s_ref[...], w_ref[...],
                         preferred_element_type=jnp.float32) + b_ref[...]


def _sys_emb(sysf, w_s, b_s):
    b, k = sysf.shape
    _, h = w_s.shape
    return pl.pallas_call(
        _sys_body,
        in_specs=[pl.BlockSpec((b, k), lambda: (0, 0)),
                  pl.BlockSpec((k, h), lambda: (0, 0)),
                  pl.BlockSpec((1, h), lambda: (0, 0))],
        out_specs=pl.BlockSpec((b, h), lambda: (0, 0)),
        out_shape=jax.ShapeDtypeStruct((b, h), jnp.float32),
    )(sysf, w_s, b_s)


def kernel(f_atoms, f_bonds, edge_index, sysf, W_i, W_h, W_o, b_o, W_s, b_s, pad_token):
    n, atom_f = f_atoms.shape
    e = f_bonds.shape[0]
    h = W_i.shape[1]
    b = sysf.shape[0]
    s = n // b

    src = edge_index[0]
    dst = edge_index[1]
    zeros = jnp.zeros((1000, h // 2), jnp.float32)
    ea = 96000 * e // 160000
    eb = e - ea

    mp_full = _make_mp_step(e, e, n, h)
    scat_a = _make_scatter_add(ea, n, h, dst_off=0, init_acc=False)
    mp_b = _make_mp_step(eb, e, n, h, dst_off=ea, init_acc=True)
    scat_b = _make_scatter_add(eb, n, h, dst_off=ea, init_acc=True)

    fbT = f_bonds.T
    msg = _matmul(fbT, W_i, bm=3200, relu=True)
    inp = _matmul(fbT, W_i, bm=3200, relu=False)

    g = mp_full(msg, dst, src, zeros)
    m1 = _fused_iter(msg, inp, g, W_h, 1600, 0, ea)
    m2 = _fused_iter(msg, inp, g, W_h, 1600, 1, ea, eb)
    accp = scat_a(m1, dst, zeros)
    g = mp_b(m2, dst, src, accp)
    m1 = _fused_iter(m1, inp, g, W_h, 1600, 0, ea)
    m2 = _fused_iter(m2, inp, g, W_h, 1600, 1, ea, eb)
    accp = scat_a(m1, dst, zeros)
    a_msg = scat_b(m2, dst, accp)

    atoms = _final_atoms(f_atoms, a_msg, W_o[:atom_f], W_o[atom_f:],
                         b_o[None, :], bm=1000)
    sys_out = _sys_emb(sysf, W_s, b_s[None, :])
    return (sys_out[:, None, :], atoms.reshape(b, s, h))
